# Initial kernel scaffold; baseline (speedup 1.0000x reference)
import jax
import jax.numpy as jnp
from jax.experimental import pallas as pl


def kernel(x, edge_index, W1, b1, W2, b2):
    raise NotImplementedError("write your pallas kernel here")


# speedup vs baseline: 15.2544x; 15.2544x over previous
import functools

import jax
import jax.numpy as jnp
from jax import lax
from jax.experimental import pallas as pl
from jax.experimental.pallas import tpu as pltpu
from jax.experimental.pallas import tpu_sc as plsc

N = 10000
E = 320000
IN_CH = 128
HID = 16

NC = 2
NS = 16
L = 16
NW = NC * NS

CH = 128
CPT = 80
EPT = CH * CPT
E_PAD = EPT * NW
N_PAD = 10240
RPT = N_PAD // NS

_mesh = plsc.VectorSubcoreMesh(
    core_axis_name="c", subcore_axis_name="s", num_cores=NC, num_subcores=NS
)

_sc_params = pltpu.CompilerParams(
    use_tc_tiling_on_sc=False, needs_layout_passes=False
)


def _worker_id():
    return lax.axis_index("s") * NC + lax.axis_index("c")


@functools.partial(
    pl.kernel,
    out_type=jax.ShapeDtypeStruct((NC, N_PAD), jnp.float32),
    mesh=_mesh,
    compiler_params=_sc_params,
    scratch_types=[
        pltpu.VMEM_SHARED((N_PAD,), jnp.float32),
        pltpu.VMEM((CH,), jnp.float32),
        pltpu.VMEM((CH,), jnp.int32),
    ],
)
def _deg_kernel(dst_hbm, out_hbm, acc_sh, buf, idxd):
    c = lax.axis_index("c")
    s = lax.axis_index("s")
    wid = _worker_id()

    for g in range(CH // L):
        buf[pl.ds(g * L, L)] = jnp.zeros((L,), jnp.float32)
    for k in range(RPT // CH):
        pltpu.sync_copy(buf, acc_sh.at[pl.ds(s * RPT + k * CH, CH)])
    plsc.subcore_barrier()

    for g in range(CH // L):
        buf[pl.ds(g * L, L)] = jnp.ones((L,), jnp.float32)

    def chunk(j, carry):
        base = wid * EPT + j * CH
        pltpu.sync_copy(dst_hbm.at[pl.ds(base, CH)], idxd)
        pltpu.sync_copy(buf, acc_sh.at[idxd], add=True)
        return carry

    lax.fori_loop(0, CPT, chunk, 0)
    plsc.subcore_barrier()
    pltpu.sync_copy(acc_sh.at[pl.ds(s * RPT, RPT)], out_hbm.at[c, pl.ds(s * RPT, RPT)])


@functools.partial(
    pl.kernel,
    out_type=jax.ShapeDtypeStruct((NC, N_PAD, HID), jnp.float32),
    mesh=_mesh,
    compiler_params=_sc_params,
    scratch_types=[
        pltpu.VMEM_SHARED((N_PAD, HID), jnp.float32),
        pltpu.VMEM((CH,), jnp.int32),
        pltpu.VMEM((CH,), jnp.int32),
        pltpu.VMEM((CH, HID), jnp.float32),
        pltpu.SemaphoreType.DMA,
    ],
)
def _scatter_kernel(g_hbm, src_hbm, dst_hbm, out_hbm, acc_sh, idxs, idxd, rows, sem):
    c = lax.axis_index("c")
    s = lax.axis_index("s")
    wid = _worker_id()

    def zrow(i, carry):
        rows[i] = jnp.zeros((L,), jnp.float32)
        return carry

    lax.fori_loop(0, CH, zrow, 0)
    for k in range(RPT // CH):
        pltpu.sync_copy(rows, acc_sh.at[pl.ds(s * RPT + k * CH, CH), :])
    plsc.subcore_barrier()

    def chunk(j, carry):
        base = wid * EPT + j * CH
        pltpu.sync_copy(src_hbm.at[pl.ds(base, CH)], idxs)
        pltpu.sync_copy(dst_hbm.at[pl.ds(base, CH)], idxd)
        pltpu.async_copy(g_hbm.at[idxs], rows, sem).wait()
        pltpu.sync_copy(rows, acc_sh.at[idxd], add=True)
        return carry

    lax.fori_loop(0, CPT, chunk, 0)
    plsc.subcore_barrier()
    pltpu.sync_copy(
        acc_sh.at[pl.ds(s * RPT, RPT), :], out_hbm.at[c, pl.ds(s * RPT, RPT), :]
    )


@functools.partial(
    pl.kernel,
    out_type=jax.ShapeDtypeStruct((E_PAD,), jnp.float32),
    mesh=_mesh,
    compiler_params=_sc_params,
    scratch_types=[
        pltpu.VMEM((CH,), jnp.int32),
        pltpu.VMEM((CH,), jnp.int32),
        pltpu.VMEM((CH, HID), jnp.float32),
        pltpu.VMEM((CH, HID), jnp.float32),
        pltpu.VMEM((CH,), jnp.float32),
        pltpu.SemaphoreType.DMA,
        pltpu.SemaphoreType.DMA,
    ],
)
def _score_kernel(h2_hbm, src_hbm, dst_hbm, out_hbm, idxs, idxd, av, bv, sv, sema, semb):
    wid = _worker_id()

    def chunk(j, carry):
        base = wid * EPT + j * CH
        pltpu.sync_copy(src_hbm.at[pl.ds(base, CH)], idxs)
        pltpu.sync_copy(dst_hbm.at[pl.ds(base, CH)], idxd)
        cpa = pltpu.async_copy(h2_hbm.at[idxs], av, sema)
        cpb = pltpu.async_copy(h2_hbm.at[idxd], bv, semb)
        cpa.wait()
        cpb.wait()
        for g in range(CH // L):
            rows_idx = lax.iota(jnp.int32, L) + g * L
            acc = jnp.zeros((L,), jnp.float32)
            for d in range(HID):
                cols = jnp.full((L,), d, jnp.int32)
                ga = plsc.load_gather(av, [rows_idx, cols])
                gb = plsc.load_gather(bv, [rows_idx, cols])
                acc = acc + ga * gb
            sv[pl.ds(g * L, L)] = 1.0 / (1.0 + jnp.exp(-acc))
        pltpu.sync_copy(sv, out_hbm.at[pl.ds(base, CH)])
        return carry

    lax.fori_loop(0, CPT, chunk, 0)


R = 2048
GRID = N_PAD // R


def _tc1_body(deg_ref, x_ref, w1_ref, dinv_ref, g1_ref):
    dsum = deg_ref[0, :] + deg_ref[1, :] + 1.0
    dinv = lax.rsqrt(dsum)
    dinv_ref[...] = dinv[:, None]
    h = jnp.dot(x_ref[...], w1_ref[...], preferred_element_type=jnp.float32)
    g1_ref[...] = h * dinv[:, None]


def _tc1(degp, xp, W1):
    return pl.pallas_call(
        _tc1_body,
        grid=(GRID,),
        in_specs=[
            pl.BlockSpec((NC, R), lambda i: (0, i)),
            pl.BlockSpec((R, IN_CH), lambda i: (i, 0)),
            pl.BlockSpec((IN_CH, HID), lambda i: (0, 0)),
        ],
        out_specs=[
            pl.BlockSpec((R, 1), lambda i: (i, 0)),
            pl.BlockSpec((R, HID), lambda i: (i, 0)),
        ],
        out_shape=[
            jax.ShapeDtypeStruct((N_PAD, 1), jnp.float32),
            jax.ShapeDtypeStruct((N_PAD, HID), jnp.float32),
        ],
    )(degp, xp, W1)


def _tc2_body(s1_ref, g1_ref, dinv_ref, b1_ref, w2_ref, g2_ref):
    tot = s1_ref[0] + s1_ref[1] + g1_ref[...]
    dinv = dinv_ref[...]
    h1 = jnp.maximum(dinv * tot + b1_ref[...], 0.0)
    g2_ref[...] = dinv * jnp.dot(h1, w2_ref[...], preferred_element_type=jnp.float32)


def _tc2(s1p, g1, dinv, b1r, W2):
    return pl.pallas_call(
        _tc2_body,
        grid=(GRID,),
        in_specs=[
            pl.BlockSpec((NC, R, HID), lambda i: (0, i, 0)),
            pl.BlockSpec((R, HID), lambda i: (i, 0)),
            pl.BlockSpec((R, 1), lambda i: (i, 0)),
            pl.BlockSpec((1, HID), lambda i: (0, 0)),
            pl.BlockSpec((HID, HID), lambda i: (0, 0)),
        ],
        out_specs=pl.BlockSpec((R, HID), lambda i: (i, 0)),
        out_shape=jax.ShapeDtypeStruct((N_PAD, HID), jnp.float32),
    )(s1p, g1, dinv, b1r, W2)


def _tc3_body(s2_ref, g2_ref, dinv_ref, b2_ref, h2_ref):
    tot = s2_ref[0] + s2_ref[1] + g2_ref[...]
    h2_ref[...] = dinv_ref[...] * tot + b2_ref[...]


def _tc3(s2p, g2, dinv, b2r):
    return pl.pallas_call(
        _tc3_body,
        grid=(GRID,),
        in_specs=[
            pl.BlockSpec((NC, R, HID), lambda i: (0, i, 0)),
            pl.BlockSpec((R, HID), lambda i: (i, 0)),
            pl.BlockSpec((R, 1), lambda i: (i, 0)),
            pl.BlockSpec((1, HID), lambda i: (0, 0)),
        ],
        out_specs=pl.BlockSpec((R, HID), lambda i: (i, 0)),
        out_shape=jax.ShapeDtypeStruct((N_PAD, HID), jnp.float32),
    )(s2p, g2, dinv, b2r)


@jax.jit
def kernel(x, edge_index, W1, b1, W2, b2):
    src = edge_index[0]
    dst = edge_index[1]
    pad_e = E_PAD - E
    srcp = jnp.concatenate([src, jnp.zeros((pad_e,), src.dtype)])
    dstp = jnp.concatenate([dst, jnp.full((pad_e,), N, dst.dtype)])
    xp = jnp.pad(x, ((0, N_PAD - N), (0, 0)))
    b1r = b1.reshape(1, HID)
    b2r = b2.reshape(1, HID)

    degp = _deg_kernel(dstp)
    dinv, g1 = _tc1(degp, xp, W1)
    s1p = _scatter_kernel(g1, srcp, dstp)
    g2 = _tc2(s1p, g1, dinv, b1r, W2)
    s2p = _scatter_kernel(g2, srcp, dstp)
    h2 = _tc3(s2p, g2, dinv, b2r)
    scores = _score_kernel(h2, srcp, dstp)
    return scores[:E]


# speedup vs baseline: 32.5491x; 2.1338x over previous
import functools

import jax
import jax.numpy as jnp
from jax import lax
from jax.experimental import pallas as pl
from jax.experimental.pallas import tpu as pltpu
from jax.experimental.pallas import tpu_sc as plsc

N = 10000
E = 320000
IN_CH = 128
HID = 16

NC = 2
NS = 16
L = 16
NW = NC * NS

CH = 128
CPT = 80
EPT = CH * CPT
E_PAD = EPT * NW
N_PAD = 10240
RPT = N_PAD // NS

_mesh = plsc.VectorSubcoreMesh(
    core_axis_name="c", subcore_axis_name="s", num_cores=NC, num_subcores=NS
)

_sc_params = pltpu.CompilerParams(
    use_tc_tiling_on_sc=False, needs_layout_passes=False
)


def _worker_id():
    return lax.axis_index("s") * NC + lax.axis_index("c")


DK = 8


@functools.partial(
    pl.kernel,
    out_type=jax.ShapeDtypeStruct((NC, N_PAD), jnp.float32),
    mesh=_mesh,
    compiler_params=_sc_params,
    scratch_types=[
        pltpu.VMEM_SHARED((N_PAD,), jnp.float32),
        pltpu.VMEM((CPT, CH), jnp.int32),
        pltpu.VMEM((CH,), jnp.float32),
        pltpu.SemaphoreType.DMA,
    ],
)
def _deg_kernel(dst_hbm, out_hbm, acc_sh, idxd, buf, sem):
    c = lax.axis_index("c")
    s = lax.axis_index("s")
    wid = _worker_id()

    pltpu.sync_copy(dst_hbm.at[wid], idxd)
    for g in range(CH // L):
        buf[pl.ds(g * L, L)] = jnp.zeros((L,), jnp.float32)
    for k in range(RPT // CH):
        pltpu.sync_copy(buf, acc_sh.at[pl.ds(s * RPT + k * CH, CH)])
    plsc.subcore_barrier()

    for g in range(CH // L):
        buf[pl.ds(g * L, L)] = jnp.ones((L,), jnp.float32)

    def group(t, carry):
        for b in range(DK):
            pltpu.async_copy(buf, acc_sh.at[idxd.at[t * DK + b]], sem, add=True)
        for b in range(DK):
            pltpu.make_async_copy(buf, acc_sh.at[idxd.at[t * DK + b]], sem).wait()
        return carry

    lax.fori_loop(0, CPT // DK, group, 0)
    plsc.subcore_barrier()
    pltpu.sync_copy(acc_sh.at[pl.ds(s * RPT, RPT)], out_hbm.at[c, pl.ds(s * RPT, RPT)])


@functools.partial(
    pl.kernel,
    out_type=jax.ShapeDtypeStruct((NC, N_PAD, HID), jnp.float32),
    mesh=_mesh,
    compiler_params=_sc_params,
    scratch_types=[
        pltpu.VMEM_SHARED((N_PAD, HID), jnp.float32),
        pltpu.VMEM((CPT, CH), jnp.int32),
        pltpu.VMEM((CPT, CH), jnp.int32),
        pltpu.VMEM((CH, HID), jnp.float32),
        pltpu.VMEM((CH, HID), jnp.float32),
        pltpu.SemaphoreType.DMA,
        pltpu.SemaphoreType.DMA,
    ],
)
def _scatter_kernel(
    g_hbm, src_hbm, dst_hbm, out_hbm, acc_sh, idxs, idxd, rows0, rows1, g0, g1
):
    c = lax.axis_index("c")
    s = lax.axis_index("s")
    wid = _worker_id()

    pltpu.sync_copy(src_hbm.at[wid], idxs)
    pltpu.sync_copy(dst_hbm.at[wid], idxd)

    def zrow(i, carry):
        rows0[i] = jnp.zeros((L,), jnp.float32)
        return carry

    lax.fori_loop(0, CH, zrow, 0)
    for k in range(RPT // CH):
        pltpu.sync_copy(rows0, acc_sh.at[pl.ds(s * RPT + k * CH, CH), :])
    plsc.subcore_barrier()

    pltpu.async_copy(g_hbm.at[idxs.at[0]], rows0, g0)
    pltpu.async_copy(g_hbm.at[idxs.at[1]], rows1, g1)

    def body(t, carry):
        j0 = 2 * t
        j1 = j0 + 1

        pltpu.make_async_copy(g_hbm.at[idxs.at[j0]], rows0, g0).wait()
        pltpu.sync_copy(rows0, acc_sh.at[idxd.at[j0]], add=True)

        @pl.when(j0 + 2 < CPT)
        def _():
            pltpu.async_copy(g_hbm.at[idxs.at[j0 + 2]], rows0, g0)

        pltpu.make_async_copy(g_hbm.at[idxs.at[j1]], rows1, g1).wait()
        pltpu.sync_copy(rows1, acc_sh.at[idxd.at[j1]], add=True)

        @pl.when(j1 + 2 < CPT)
        def _():
            pltpu.async_copy(g_hbm.at[idxs.at[j1 + 2]], rows1, g1)

        return carry

    lax.fori_loop(0, CPT // 2, body, 0)
    plsc.subcore_barrier()
    pltpu.sync_copy(
        acc_sh.at[pl.ds(s * RPT, RPT), :], out_hbm.at[c, pl.ds(s * RPT, RPT), :]
    )


@functools.partial(
    pl.kernel,
    out_type=jax.ShapeDtypeStruct((E_PAD,), jnp.float32),
    mesh=_mesh,
    compiler_params=_sc_params,
    scratch_types=[
        pltpu.VMEM((CPT, CH), jnp.int32),
        pltpu.VMEM((CPT, CH), jnp.int32),
        pltpu.VMEM((CH, HID), jnp.float32),
        pltpu.VMEM((CH, HID), jnp.float32),
        pltpu.VMEM((CH, HID), jnp.float32),
        pltpu.VMEM((CH, HID), jnp.float32),
        pltpu.VMEM((CH,), jnp.float32),
        pltpu.VMEM((CH,), jnp.float32),
        pltpu.SemaphoreType.DMA,
        pltpu.SemaphoreType.DMA,
        pltpu.SemaphoreType.DMA,
        pltpu.SemaphoreType.DMA,
        pltpu.SemaphoreType.DMA,
        pltpu.SemaphoreType.DMA,
    ],
)
def _score_kernel(
    h2_hbm, src_hbm, dst_hbm, out_hbm,
    idxs, idxd, av0, bv0, av1, bv1, sv0, sv1,
    ga0, gb0, ga1, gb1, ws0, ws1,
):
    wid = _worker_id()

    pltpu.sync_copy(src_hbm.at[wid], idxs)
    pltpu.sync_copy(dst_hbm.at[wid], idxd)

    pltpu.async_copy(h2_hbm.at[idxs.at[0]], av0, ga0)
    pltpu.async_copy(h2_hbm.at[idxd.at[0]], bv0, gb0)
    pltpu.async_copy(h2_hbm.at[idxs.at[1]], av1, ga1)
    pltpu.async_copy(h2_hbm.at[idxd.at[1]], bv1, gb1)

    def dots(av, bv, sv):
        for g in range(CH // L):
            rows_idx = lax.iota(jnp.int32, L) + g * L
            acc = jnp.zeros((L,), jnp.float32)
            for d in range(HID):
                cols = jnp.full((L,), d, jnp.int32)
                ga = plsc.load_gather(av, [rows_idx, cols])
                gb = plsc.load_gather(bv, [rows_idx, cols])
                acc = acc + ga * gb
            sv[pl.ds(g * L, L)] = 1.0 / (1.0 + jnp.exp(-acc))

    def half(t, j, idxa_row, idxb_row, av, bv, sv, ga, gb, ws):
        base = wid * EPT + j * CH
        pltpu.make_async_copy(h2_hbm.at[idxa_row], av, ga).wait()
        pltpu.make_async_copy(h2_hbm.at[idxb_row], bv, gb).wait()

        @pl.when(t > 0)
        def _():
            pltpu.make_async_copy(
                sv, out_hbm.at[pl.ds(base - 2 * CH, CH)], ws
            ).wait()

        dots(av, bv, sv)
        pltpu.async_copy(sv, out_hbm.at[pl.ds(base, CH)], ws)

        @pl.when(j + 2 < CPT)
        def _():
            pltpu.async_copy(h2_hbm.at[idxs.at[j + 2]], av, ga)
            pltpu.async_copy(h2_hbm.at[idxd.at[j + 2]], bv, gb)

    def body(t, carry):
        j0 = 2 * t
        j1 = j0 + 1
        half(t, j0, idxs.at[j0], idxd.at[j0], av0, bv0, sv0, ga0, gb0, ws0)
        half(t, j1, idxs.at[j1], idxd.at[j1], av1, bv1, sv1, ga1, gb1, ws1)
        return carry

    lax.fori_loop(0, CPT // 2, body, 0)
    pltpu.make_async_copy(sv0, out_hbm.at[pl.ds(wid * EPT, CH)], ws0).wait()
    pltpu.make_async_copy(sv1, out_hbm.at[pl.ds(wid * EPT, CH)], ws1).wait()


R = 2048
GRID = N_PAD // R


def _tc1_body(deg_ref, x_ref, w1_ref, dinv_ref, g1_ref):
    dsum = deg_ref[0, :] + deg_ref[1, :] + 1.0
    dinv = lax.rsqrt(dsum)
    dinv_ref[...] = dinv[:, None]
    h = jnp.dot(x_ref[...], w1_ref[...], preferred_element_type=jnp.float32)
    g1_ref[...] = h * dinv[:, None]


def _tc1(degp, xp, W1):
    return pl.pallas_call(
        _tc1_body,
        grid=(GRID,),
        in_specs=[
            pl.BlockSpec((NC, R), lambda i: (0, i)),
            pl.BlockSpec((R, IN_CH), lambda i: (i, 0)),
            pl.BlockSpec((IN_CH, HID), lambda i: (0, 0)),
        ],
        out_specs=[
            pl.BlockSpec((R, 1), lambda i: (i, 0)),
            pl.BlockSpec((R, HID), lambda i: (i, 0)),
        ],
        out_shape=[
            jax.ShapeDtypeStruct((N_PAD, 1), jnp.float32),
            jax.ShapeDtypeStruct((N_PAD, HID), jnp.float32),
        ],
    )(degp, xp, W1)


def _tc2_body(s1_ref, g1_ref, dinv_ref, b1_ref, w2_ref, g2_ref):
    tot = s1_ref[0] + s1_ref[1] + g1_ref[...]
    dinv = dinv_ref[...]
    h1 = jnp.maximum(dinv * tot + b1_ref[...], 0.0)
    g2_ref[...] = dinv * jnp.dot(h1, w2_ref[...], preferred_element_type=jnp.float32)


def _tc2(s1p, g1, dinv, b1r, W2):
    return pl.pallas_call(
        _tc2_body,
        grid=(GRID,),
        in_specs=[
            pl.BlockSpec((NC, R, HID), lambda i: (0, i, 0)),
            pl.BlockSpec((R, HID), lambda i: (i, 0)),
            pl.BlockSpec((R, 1), lambda i: (i, 0)),
            pl.BlockSpec((1, HID), lambda i: (0, 0)),
            pl.BlockSpec((HID, HID), lambda i: (0, 0)),
        ],
        out_specs=pl.BlockSpec((R, HID), lambda i: (i, 0)),
        out_shape=jax.ShapeDtypeStruct((N_PAD, HID), jnp.float32),
    )(s1p, g1, dinv, b1r, W2)


def _tc3_body(s2_ref, g2_ref, dinv_ref, b2_ref, h2_ref):
    tot = s2_ref[0] + s2_ref[1] + g2_ref[...]
    h2_ref[...] = dinv_ref[...] * tot + b2_ref[...]


def _tc3(s2p, g2, dinv, b2r):
    return pl.pallas_call(
        _tc3_body,
        grid=(GRID,),
        in_specs=[
            pl.BlockSpec((NC, R, HID), lambda i: (0, i, 0)),
            pl.BlockSpec((R, HID), lambda i: (i, 0)),
            pl.BlockSpec((R, 1), lambda i: (i, 0)),
            pl.BlockSpec((1, HID), lambda i: (0, 0)),
        ],
        out_specs=pl.BlockSpec((R, HID), lambda i: (i, 0)),
        out_shape=jax.ShapeDtypeStruct((N_PAD, HID), jnp.float32),
    )(s2p, g2, dinv, b2r)


@jax.jit
def kernel(x, edge_index, W1, b1, W2, b2):
    src = edge_index[0]
    dst = edge_index[1]
    pad_e = E_PAD - E
    srcp = jnp.concatenate([src, jnp.zeros((pad_e,), src.dtype)])
    dstp = jnp.concatenate([dst, jnp.full((pad_e,), N, dst.dtype)])
    src3 = srcp.reshape(NW, CPT, CH)
    dst3 = dstp.reshape(NW, CPT, CH)
    xp = jnp.pad(x, ((0, N_PAD - N), (0, 0)))
    b1r = b1.reshape(1, HID)
    b2r = b2.reshape(1, HID)

    degp = _deg_kernel(dst3)
    dinv, g1 = _tc1(degp, xp, W1)
    s1p = _scatter_kernel(g1, src3, dst3)
    g2 = _tc2(s1p, g1, dinv, b1r, W2)
    s2p = _scatter_kernel(g2, src3, dst3)
    h2 = _tc3(s2p, g2, dinv, b2r)
    scores = _score_kernel(h2, src3, dst3)
    return scores[:E]
